# baseline (speedup 1.0000x reference)
import functools

import jax
import jax.numpy as jnp
from jax import lax
from jax.experimental import pallas as pl
from jax.experimental.pallas import tpu as pltpu
from jax.experimental.pallas import tpu_sc as plsc

BATCH = 4
SEQ = 4096
DIM = 1024

_NC = 2
_NS = 16
_NW = _NC * _NS

_CH_ROWS = 16
_CH = _CH_ROWS * DIM
_ROWS_PER_W = SEQ // _NW


def _make_sc_add():
    mesh = plsc.VectorSubcoreMesh(core_axis_name="c", subcore_axis_name="s")

    @functools.partial(
        pl.kernel,
        mesh=mesh,
        out_type=jax.ShapeDtypeStruct((BATCH, SEQ * DIM), jnp.float32),
        scratch_types=[
            pltpu.VMEM((_CH,), jnp.float32),
            pltpu.VMEM((_CH,), jnp.float32),
        ],
    )
    def sc_add(in_hbm, pos_hbm, out_hbm, pos_v, io_v):
        wid = lax.axis_index("s") * _NC + lax.axis_index("c")
        base = wid * _ROWS_PER_W * DIM

        def chunk_body(ci, _):
            off = base + ci * _CH
            pltpu.sync_copy(pos_hbm.at[pl.ds(off, _CH)], pos_v)

            def batch_body(b, _):
                pltpu.sync_copy(in_hbm.at[b, pl.ds(off, _CH)], io_v)

                def add_body(i, _):
                    s = pl.ds(i * 16, 16)
                    io_v[s] = io_v[s] + pos_v[s]
                    return 0

                lax.fori_loop(0, _CH // 16, add_body, 0)
                pltpu.sync_copy(io_v, out_hbm.at[b, pl.ds(off, _CH)])
                return 0

            lax.fori_loop(0, BATCH, batch_body, 0)
            return 0

        lax.fori_loop(0, _ROWS_PER_W // _CH_ROWS, chunk_body, 0)

    return sc_add


_sc_add = _make_sc_add()


def kernel(inputs, pos_table):
    batch, seq, dim = inputs.shape
    out = _sc_add(inputs.reshape(batch, seq * dim), pos_table.reshape(seq * dim))
    return out.reshape(batch, seq, dim)


# speedup vs baseline: 1.6916x; 1.6916x over previous
import functools

import jax
import jax.numpy as jnp
from jax import lax
from jax.experimental import pallas as pl
from jax.experimental.pallas import tpu as pltpu
from jax.experimental.pallas import tpu_sc as plsc

BATCH = 4
SEQ = 4096
DIM = 1024

_NC = 2
_NS = 16
_NW = _NC * _NS

_CH_ROWS = 16
_CH = _CH_ROWS * DIM
_ROWS_PER_W = SEQ // _NW
_NCHUNK = _ROWS_PER_W // _CH_ROWS


def _make_sc_add():
    mesh = plsc.VectorSubcoreMesh(core_axis_name="c", subcore_axis_name="s")

    @functools.partial(
        pl.kernel,
        mesh=mesh,
        out_type=jax.ShapeDtypeStruct((BATCH, SEQ * DIM), jnp.float32),
        scratch_types=[
            pltpu.VMEM((_CH,), jnp.float32),
            pltpu.VMEM((_CH,), jnp.float32),
            pltpu.VMEM((_CH,), jnp.float32),
            pltpu.SemaphoreType.DMA,
            pltpu.SemaphoreType.DMA,
            pltpu.SemaphoreType.DMA,
            pltpu.SemaphoreType.DMA,
        ],
    )
    def sc_add(in_hbm, pos_hbm, out_hbm, pos_v, io0, io1, si0, si1, so0, so1):
        wid = lax.axis_index("s") * _NC + lax.axis_index("c")
        base = wid * _ROWS_PER_W * DIM

        io = (io0, io1)
        sin = (si0, si1)
        sout = (so0, so1)
        steps = [(ci, b) for ci in range(_NCHUNK) for b in range(BATCH)]
        nst = len(steps)

        def in_load(t):
            ci, b = steps[t]
            off = base + ci * _CH
            return pltpu.async_copy(in_hbm.at[b, pl.ds(off, _CH)], io[t % 2], sin[t % 2])

        load_h = {0: in_load(0)}
        store_h = {}

        for t in range(nst):
            ci, b = steps[t]
            off = base + ci * _CH
            buf = t % 2
            if t + 1 < nst:
                if t >= 1:
                    store_h[t - 1].wait()
                load_h[t + 1] = in_load(t + 1)
            if b == 0:
                pltpu.sync_copy(pos_hbm.at[pl.ds(off, _CH)], pos_v)
            load_h[t].wait()
            io_ref = io[buf]

            def add_body(i, _):
                ib = i * 128
                for u in range(8):
                    s = pl.ds(ib + u * 16, 16)
                    io_ref[s] = io_ref[s] + pos_v[s]
                return 0

            lax.fori_loop(0, _CH // 128, add_body, 0)
            store_h[t] = pltpu.async_copy(io_ref, out_hbm.at[b, pl.ds(off, _CH)], sout[buf])

        store_h[nst - 2].wait()
        store_h[nst - 1].wait()

    return sc_add


_sc_add = _make_sc_add()


def kernel(inputs, pos_table):
    batch, seq, dim = inputs.shape
    out = _sc_add(inputs.reshape(batch, seq * dim), pos_table.reshape(seq * dim))
    return out.reshape(batch, seq, dim)
